# baseline (speedup 1.0000x reference)
import functools

import jax
import jax.numpy as jnp
from jax import lax
from jax.experimental import pallas as pl
from jax.experimental.pallas import tpu as pltpu
from jax.experimental.pallas import tpu_sc as plsc

N = 10000
E = 320000
D = 128
NP = 10240
EP = 327680
NW = 32
CHUNKS = EP // (NW * 128)
RPT = NP // 16

def _mesh():
    return plsc.VectorSubcoreMesh(core_axis_name="c", subcore_axis_name="s")


@functools.cache
def _make_deg_sc():
  return functools.partial(
    pl.kernel,
    out_type=jax.ShapeDtypeStruct((NW, NP // 128, 128), jnp.float32),
    mesh=_mesh(),
    scratch_types=[
        pltpu.VMEM((CHUNKS, 128), jnp.int32),
        pltpu.VMEM((NP // 128, 128), jnp.float32),
    ],
    compiler_params=pltpu.CompilerParams(needs_layout_passes=False),
  )(_deg_sc)


def _deg_sc(dst_hbm, out_hbm, idx_v, deg_v):
    c = lax.axis_index("c")
    s = lax.axis_index("s")
    wid = c * 16 + s
    pltpu.sync_copy(dst_hbm.at[pl.ds(wid * CHUNKS, CHUNKS)], idx_v)
    z16 = jnp.zeros((16,), jnp.float32)

    def zero(j, _):
        for k in range(8):
            deg_v[j, pl.ds(k * 16, 16)] = z16
        return 0

    lax.fori_loop(0, NP // 128, zero, 0)
    o16 = jnp.ones((16,), jnp.float32)

    def body(j, _):
        for k in range(8):
            di = idx_v[j, pl.ds(k * 16, 16)]
            plsc.addupdate_scatter(deg_v, [di >> 7, di & 127], o16)
        return 0

    lax.fori_loop(0, CHUNKS, body, 0)
    pltpu.sync_copy(deg_v, out_hbm.at[wid])


TCH = EP // (16 * 128)
ER = EP // 128 + TCH
_PS = 32
C0 = 32


@functools.cache
def _make_scatter_sc():
  return functools.partial(
    pl.kernel,
    out_type=jax.ShapeDtypeStruct((2, NP, D), jnp.float32),
    mesh=_mesh(),
    scratch_types=[
        pltpu.VMEM((_PS, 128), jnp.int32),
        pltpu.VMEM((_PS, 128), jnp.int32),
        pltpu.VMEM((256, D), jnp.float32),
        pltpu.SemaphoreType.DMA((2,)),
        pltpu.VMEM_SHARED((NP, D), jnp.float32),
    ],
    compiler_params=pltpu.CompilerParams(needs_layout_passes=False),
  )(_scatter_sc)


def _pipe(g_hbm, src_hbm, dst_hbm, isv, idv, buf, gsem, acc_sp, base0, nchunks):
    for p in range(nchunks // _PS):
        wbase = base0 + p * _PS
        pltpu.sync_copy(src_hbm.at[pl.ds(wbase, _PS)], isv)
        pltpu.sync_copy(dst_hbm.at[pl.ds(wbase, _PS)], idv)
        for r in range(2):
            pltpu.async_copy(g_hbm.at[isv.at[r]], buf.at[pl.ds(r * 128, 128)],
                             gsem.at[r])

        def body(j, _):
            half = (j % 2) * 128
            bref = buf.at[pl.ds(half, 128)]
            pltpu.make_async_copy(g_hbm.at[pl.ds(0, 128)], bref,
                                  gsem.at[j % 2]).wait()
            pltpu.sync_copy(bref, acc_sp.at[idv.at[j - 2]], add=True)

            @pl.when(j < _PS)
            def _fire():
                pltpu.async_copy(g_hbm.at[isv.at[j]], bref, gsem.at[j % 2])

            return 0

        lax.fori_loop(2, _PS + 2, body, 0)


def _scatter_sc(g_hbm, src_hbm, dst_hbm, zeros_hbm, out_hbm, isv, idv,
                buf, gsem, acc_sp):
    c = lax.axis_index("c")
    s = lax.axis_index("s")
    pltpu.sync_copy(zeros_hbm, buf.at[pl.ds(0, 128)])
    for k in range(RPT // 128):
        pltpu.sync_copy(buf.at[pl.ds(0, 128)],
                        acc_sp.at[pl.ds(s * RPT + k * 128, 128)])
    plsc.subcore_barrier()

    @pl.when(c == 0)
    def _core0():
        _pipe(g_hbm, src_hbm, dst_hbm, isv, idv, buf, gsem, acc_sp,
              s * TCH, C0)

    @pl.when(c == 1)
    def _core1():
        _pipe(g_hbm, src_hbm, dst_hbm, isv, idv, buf, gsem, acc_sp,
              s * TCH + C0, TCH - C0)

    plsc.subcore_barrier()
    for k in range(RPT // 128):
        pltpu.sync_copy(acc_sp.at[pl.ds(s * RPT + k * 128, 128)],
                        buf.at[pl.ds(0, 128)])
        pltpu.sync_copy(buf.at[pl.ds(0, 128)],
                        out_hbm.at[c, pl.ds(s * RPT + k * 128, 128)])


@functools.cache
def _make_edge_sc():
  return functools.partial(
    pl.kernel,
    out_type=jax.ShapeDtypeStruct((EP // 128, 128), jnp.float32),
    mesh=_mesh(),
    scratch_types=[
        pltpu.VMEM((CHUNKS, 128), jnp.int32),
        pltpu.VMEM((CHUNKS, 128), jnp.int32),
        pltpu.VMEM((NP // 128, 128), jnp.float32),
        pltpu.VMEM((CHUNKS, 128), jnp.float32),
    ],
    compiler_params=pltpu.CompilerParams(needs_layout_passes=False),
  )(_edge_sc)


def _edge_sc(z_hbm, src_hbm, dst_hbm, out_hbm, isv, idv, z_v, y_v):
    c = lax.axis_index("c")
    s = lax.axis_index("s")
    wid = c * 16 + s
    pltpu.sync_copy(z_hbm, z_v)
    pltpu.sync_copy(src_hbm.at[pl.ds(wid * CHUNKS, CHUNKS)], isv)
    pltpu.sync_copy(dst_hbm.at[pl.ds(wid * CHUNKS, CHUNKS)], idv)

    def body(j, _):
        for k in range(8):
            si = isv[j, pl.ds(k * 16, 16)]
            di = idv[j, pl.ds(k * 16, 16)]
            zs = plsc.load_gather(z_v, [si // 128, si % 128])
            zd = plsc.load_gather(z_v, [di // 128, di % 128])
            y_v[j, pl.ds(k * 16, 16)] = (zs + zd) * 0.5
        return 0

    lax.fori_loop(0, CHUNKS, body, 0)
    pltpu.sync_copy(y_v, out_hbm.at[pl.ds(wid * CHUNKS, CHUNKS)])


_BR = 1280

def _mm1_body(x_ref, w_ref, deg_ref, g_ref, dinv_ref):
    deg = jnp.sum(deg_ref[...], axis=0) + 1.0
    dinv = lax.rsqrt(deg)[:, None]
    h = jnp.dot(x_ref[...], w_ref[...], precision=lax.Precision.HIGHEST,
                preferred_element_type=jnp.float32)
    g_ref[...] = dinv * h
    dinv_ref[...] = dinv


def _mm1_tc(x_p, W1, deg_part):
    return pl.pallas_call(
        _mm1_body,
        grid=(NP // _BR,),
        in_specs=[
            pl.BlockSpec((_BR, D), lambda i: (i, 0)),
            pl.BlockSpec((D, D), lambda i: (0, 0)),
            pl.BlockSpec((NW, _BR), lambda i: (0, i)),
        ],
        out_specs=[
            pl.BlockSpec((_BR, D), lambda i: (i, 0)),
            pl.BlockSpec((_BR, 1), lambda i: (i, 0)),
        ],
        out_shape=[
            jax.ShapeDtypeStruct((NP, D), jnp.float32),
            jax.ShapeDtypeStruct((NP, 1), jnp.float32),
        ],
    )(x_p, W1, deg_part)


def _mm2_body(acc_ref, g_ref, dinv_ref, b_ref, w_ref, g2_ref):
    a = acc_ref[0] + acc_ref[1] + g_ref[...]
    h = jnp.maximum(dinv_ref[...] * a + b_ref[...], 0.0)
    g2_ref[...] = dinv_ref[...] * jnp.dot(
        h, w_ref[...], precision=lax.Precision.HIGHEST,
        preferred_element_type=jnp.float32)


def _mm2_tc(acc, g1, dinv, b1, W2):
    return pl.pallas_call(
        _mm2_body,
        grid=(NP // _BR,),
        in_specs=[
            pl.BlockSpec((2, _BR, D), lambda i: (0, i, 0)),
            pl.BlockSpec((_BR, D), lambda i: (i, 0)),
            pl.BlockSpec((_BR, 1), lambda i: (i, 0)),
            pl.BlockSpec((1, D), lambda i: (0, 0)),
            pl.BlockSpec((D, D), lambda i: (0, 0)),
        ],
        out_specs=pl.BlockSpec((_BR, D), lambda i: (i, 0)),
        out_shape=jax.ShapeDtypeStruct((NP, D), jnp.float32),
    )(acc, g1, dinv, b1.reshape(1, D), W2)


def _mm3_body(acc_ref, g_ref, dinv_ref, b_ref, we_ref, be_ref, h_ref, z_ref):
    a = acc_ref[0] + acc_ref[1] + g_ref[...]
    h = jnp.maximum(dinv_ref[...] * a + b_ref[...], 0.0)
    h_ref[...] = h
    z_ref[...] = jnp.dot(h, we_ref[...], precision=lax.Precision.HIGHEST,
                         preferred_element_type=jnp.float32) + be_ref[...]


def _mm3_tc(acc, g2, dinv, b2, We, be):
    return pl.pallas_call(
        _mm3_body,
        grid=(NP // _BR,),
        in_specs=[
            pl.BlockSpec((2, _BR, D), lambda i: (0, i, 0)),
            pl.BlockSpec((_BR, D), lambda i: (i, 0)),
            pl.BlockSpec((_BR, 1), lambda i: (i, 0)),
            pl.BlockSpec((1, D), lambda i: (0, 0)),
            pl.BlockSpec((D, 1), lambda i: (0, 0)),
            pl.BlockSpec((1, 1), lambda i: (0, 0)),
        ],
        out_specs=[
            pl.BlockSpec((_BR, D), lambda i: (i, 0)),
            pl.BlockSpec((_BR, 1), lambda i: (i, 0)),
        ],
        out_shape=[
            jax.ShapeDtypeStruct((NP, D), jnp.float32),
            jax.ShapeDtypeStruct((NP, 1), jnp.float32),
        ],
    )(acc, g2, dinv, b2.reshape(1, D), We, be.reshape(1, 1))


def kernel(x, edge_index, W1, b1, W2, b2, We, be):
    src = edge_index[0].astype(jnp.int32)
    dst = edge_index[1].astype(jnp.int32)
    pad = jnp.full((ER * 128 - E,), N, dtype=jnp.int32)
    src_r = jnp.concatenate([src, pad]).reshape(ER, 128)
    dst_r = jnp.concatenate([dst, pad]).reshape(ER, 128)
    x_p = jnp.concatenate([x, jnp.zeros((NP - N, D), x.dtype)])
    zeros128 = jnp.zeros((128, D), jnp.float32)

    deg_part = _make_deg_sc()(dst_r).reshape(NW, NP)
    g1, dinv = _mm1_tc(x_p, W1, deg_part)
    acc1 = _make_scatter_sc()(g1, src_r, dst_r, zeros128)
    g2 = _mm2_tc(acc1, g1, dinv, b1, W2)
    acc2 = _make_scatter_sc()(g2, src_r, dst_r, zeros128)
    h, z = _mm3_tc(acc2, g2, dinv, b2, We, be)
    y = _make_edge_sc()(z.reshape(NP // 128, 128), src_r, dst_r)
    return (h[:N], y.reshape(-1)[:E, None])


# speedup vs baseline: 1.3306x; 1.3306x over previous
import functools

import jax
import jax.numpy as jnp
from jax import lax
from jax.experimental import pallas as pl
from jax.experimental.pallas import tpu as pltpu
from jax.experimental.pallas import tpu_sc as plsc

N = 10000
E = 320000
D = 128
NP = 10240
EP = 327680
NW = 32
CHUNKS = EP // (NW * 128)
RPT = NP // 16

def _mesh():
    return plsc.VectorSubcoreMesh(core_axis_name="c", subcore_axis_name="s")


@functools.cache
def _make_deg_sc():
  return functools.partial(
    pl.kernel,
    out_type=jax.ShapeDtypeStruct((NW, NP // 128, 128), jnp.float32),
    mesh=_mesh(),
    scratch_types=[
        pltpu.VMEM((CHUNKS, 128), jnp.int32),
        pltpu.VMEM((NP // 128, 128), jnp.float32),
    ],
    compiler_params=pltpu.CompilerParams(needs_layout_passes=False),
  )(_deg_sc)


def _deg_sc(dst_hbm, out_hbm, idx_v, deg_v):
    c = lax.axis_index("c")
    s = lax.axis_index("s")
    wid = c * 16 + s
    pltpu.sync_copy(dst_hbm.at[pl.ds(wid * CHUNKS, CHUNKS)], idx_v)
    z16 = jnp.zeros((16,), jnp.float32)

    def zero(j, _):
        for k in range(8):
            deg_v[j, pl.ds(k * 16, 16)] = z16
        return 0

    lax.fori_loop(0, NP // 128, zero, 0)
    o16 = jnp.ones((16,), jnp.float32)

    def body(j, _):
        for k in range(8):
            di = idx_v[j, pl.ds(k * 16, 16)]
            plsc.addupdate_scatter(deg_v, [di >> 7, di & 127], o16)
        return 0

    lax.fori_loop(0, CHUNKS, body, 0)
    pltpu.sync_copy(deg_v, out_hbm.at[wid])


TCH = EP // (16 * 128)
ER = EP // 128 + TCH
_PS = 32
C0 = 128


@functools.cache
def _make_scatter_sc():
  return functools.partial(
    pl.kernel,
    out_type=jax.ShapeDtypeStruct((2, NP, D), jnp.float32),
    mesh=_mesh(),
    scratch_types=[
        pltpu.VMEM((_PS, 128), jnp.int32),
        pltpu.VMEM((_PS, 128), jnp.int32),
        pltpu.VMEM((256, D), jnp.float32),
        pltpu.SemaphoreType.DMA((2,)),
        pltpu.VMEM_SHARED((NP, D), jnp.float32),
    ],
    compiler_params=pltpu.CompilerParams(needs_layout_passes=False),
  )(_scatter_sc)


def _pipe(g_hbm, src_hbm, dst_hbm, isv, idv, buf, gsem, acc_sp, base0, nchunks):
    for p in range(nchunks // _PS):
        wbase = base0 + p * _PS
        pltpu.sync_copy(src_hbm.at[pl.ds(wbase, _PS)], isv)
        pltpu.sync_copy(dst_hbm.at[pl.ds(wbase, _PS)], idv)
        for r in range(2):
            pltpu.async_copy(g_hbm.at[isv.at[r]], buf.at[pl.ds(r * 128, 128)],
                             gsem.at[r])

        def body(j, _):
            half = (j % 2) * 128
            bref = buf.at[pl.ds(half, 128)]
            pltpu.make_async_copy(g_hbm.at[pl.ds(0, 128)], bref,
                                  gsem.at[j % 2]).wait()
            pltpu.sync_copy(bref, acc_sp.at[idv.at[j - 2]], add=True)

            @pl.when(j < _PS)
            def _fire():
                pltpu.async_copy(g_hbm.at[isv.at[j]], bref, gsem.at[j % 2])

            return 0

        lax.fori_loop(2, _PS + 2, body, 0)


def _scatter_sc(g_hbm, src_hbm, dst_hbm, zeros_hbm, out_hbm, isv, idv,
                buf, gsem, acc_sp):
    c = lax.axis_index("c")
    s = lax.axis_index("s")
    pltpu.sync_copy(zeros_hbm, buf.at[pl.ds(0, 128)])
    for k in range(RPT // 128):
        pltpu.sync_copy(buf.at[pl.ds(0, 128)],
                        acc_sp.at[pl.ds(s * RPT + k * 128, 128)])
    plsc.subcore_barrier()

    @pl.when(c == 0)
    def _core0():
        _pipe(g_hbm, src_hbm, dst_hbm, isv, idv, buf, gsem, acc_sp,
              s * TCH, C0)

    @pl.when(c == 1)
    def _core1():
        _pipe(g_hbm, src_hbm, dst_hbm, isv, idv, buf, gsem, acc_sp,
              s * TCH + C0, TCH - C0)

    plsc.subcore_barrier()
    for k in range(RPT // 128):
        pltpu.sync_copy(acc_sp.at[pl.ds(s * RPT + k * 128, 128)],
                        buf.at[pl.ds(0, 128)])
        pltpu.sync_copy(buf.at[pl.ds(0, 128)],
                        out_hbm.at[c, pl.ds(s * RPT + k * 128, 128)])


@functools.cache
def _make_edge_sc():
  return functools.partial(
    pl.kernel,
    out_type=jax.ShapeDtypeStruct((EP // 128, 128), jnp.float32),
    mesh=_mesh(),
    scratch_types=[
        pltpu.VMEM((CHUNKS, 128), jnp.int32),
        pltpu.VMEM((CHUNKS, 128), jnp.int32),
        pltpu.VMEM((NP // 128, 128), jnp.float32),
        pltpu.VMEM((CHUNKS, 128), jnp.float32),
    ],
    compiler_params=pltpu.CompilerParams(needs_layout_passes=False),
  )(_edge_sc)


def _edge_sc(z_hbm, src_hbm, dst_hbm, out_hbm, isv, idv, z_v, y_v):
    c = lax.axis_index("c")
    s = lax.axis_index("s")
    wid = c * 16 + s
    pltpu.sync_copy(z_hbm, z_v)
    pltpu.sync_copy(src_hbm.at[pl.ds(wid * CHUNKS, CHUNKS)], isv)
    pltpu.sync_copy(dst_hbm.at[pl.ds(wid * CHUNKS, CHUNKS)], idv)

    def body(j, _):
        for k in range(8):
            si = isv[j, pl.ds(k * 16, 16)]
            di = idv[j, pl.ds(k * 16, 16)]
            zs = plsc.load_gather(z_v, [si // 128, si % 128])
            zd = plsc.load_gather(z_v, [di // 128, di % 128])
            y_v[j, pl.ds(k * 16, 16)] = (zs + zd) * 0.5
        return 0

    lax.fori_loop(0, CHUNKS, body, 0)
    pltpu.sync_copy(y_v, out_hbm.at[pl.ds(wid * CHUNKS, CHUNKS)])


_BR = 1280

def _mm1_body(x_ref, w_ref, deg_ref, g_ref, dinv_ref):
    deg = jnp.sum(deg_ref[...], axis=0) + 1.0
    dinv = lax.rsqrt(deg)[:, None]
    h = jnp.dot(x_ref[...], w_ref[...], precision=lax.Precision.HIGHEST,
                preferred_element_type=jnp.float32)
    g_ref[...] = dinv * h
    dinv_ref[...] = dinv


def _mm1_tc(x_p, W1, deg_part):
    return pl.pallas_call(
        _mm1_body,
        grid=(NP // _BR,),
        in_specs=[
            pl.BlockSpec((_BR, D), lambda i: (i, 0)),
            pl.BlockSpec((D, D), lambda i: (0, 0)),
            pl.BlockSpec((NW, _BR), lambda i: (0, i)),
        ],
        out_specs=[
            pl.BlockSpec((_BR, D), lambda i: (i, 0)),
            pl.BlockSpec((_BR, 1), lambda i: (i, 0)),
        ],
        out_shape=[
            jax.ShapeDtypeStruct((NP, D), jnp.float32),
            jax.ShapeDtypeStruct((NP, 1), jnp.float32),
        ],
    )(x_p, W1, deg_part)


def _mm2_body(acc_ref, g_ref, dinv_ref, b_ref, w_ref, g2_ref):
    a = acc_ref[0] + acc_ref[1] + g_ref[...]
    h = jnp.maximum(dinv_ref[...] * a + b_ref[...], 0.0)
    g2_ref[...] = dinv_ref[...] * jnp.dot(
        h, w_ref[...], precision=lax.Precision.HIGHEST,
        preferred_element_type=jnp.float32)


def _mm2_tc(acc, g1, dinv, b1, W2):
    return pl.pallas_call(
        _mm2_body,
        grid=(NP // _BR,),
        in_specs=[
            pl.BlockSpec((2, _BR, D), lambda i: (0, i, 0)),
            pl.BlockSpec((_BR, D), lambda i: (i, 0)),
            pl.BlockSpec((_BR, 1), lambda i: (i, 0)),
            pl.BlockSpec((1, D), lambda i: (0, 0)),
            pl.BlockSpec((D, D), lambda i: (0, 0)),
        ],
        out_specs=pl.BlockSpec((_BR, D), lambda i: (i, 0)),
        out_shape=jax.ShapeDtypeStruct((NP, D), jnp.float32),
    )(acc, g1, dinv, b1.reshape(1, D), W2)


def _mm3_body(acc_ref, g_ref, dinv_ref, b_ref, we_ref, be_ref, h_ref, z_ref):
    a = acc_ref[0] + acc_ref[1] + g_ref[...]
    h = jnp.maximum(dinv_ref[...] * a + b_ref[...], 0.0)
    h_ref[...] = h
    z_ref[...] = jnp.dot(h, we_ref[...], precision=lax.Precision.HIGHEST,
                         preferred_element_type=jnp.float32) + be_ref[...]


def _mm3_tc(acc, g2, dinv, b2, We, be):
    return pl.pallas_call(
        _mm3_body,
        grid=(NP // _BR,),
        in_specs=[
            pl.BlockSpec((2, _BR, D), lambda i: (0, i, 0)),
            pl.BlockSpec((_BR, D), lambda i: (i, 0)),
            pl.BlockSpec((_BR, 1), lambda i: (i, 0)),
            pl.BlockSpec((1, D), lambda i: (0, 0)),
            pl.BlockSpec((D, 1), lambda i: (0, 0)),
            pl.BlockSpec((1, 1), lambda i: (0, 0)),
        ],
        out_specs=[
            pl.BlockSpec((_BR, D), lambda i: (i, 0)),
            pl.BlockSpec((_BR, 1), lambda i: (i, 0)),
        ],
        out_shape=[
            jax.ShapeDtypeStruct((NP, D), jnp.float32),
            jax.ShapeDtypeStruct((NP, 1), jnp.float32),
        ],
    )(acc, g2, dinv, b2.reshape(1, D), We, be.reshape(1, 1))


def kernel(x, edge_index, W1, b1, W2, b2, We, be):
    src = edge_index[0].astype(jnp.int32)
    dst = edge_index[1].astype(jnp.int32)
    pad = jnp.full((ER * 128 - E,), N, dtype=jnp.int32)
    src_r = jnp.concatenate([src, pad]).reshape(ER, 128)
    dst_r = jnp.concatenate([dst, pad]).reshape(ER, 128)
    x_p = jnp.concatenate([x, jnp.zeros((NP - N, D), x.dtype)])
    zeros128 = jnp.zeros((128, D), jnp.float32)

    deg_part = _make_deg_sc()(dst_r).reshape(NW, NP)
    g1, dinv = _mm1_tc(x_p, W1, deg_part)
    acc1 = _make_scatter_sc()(g1, src_r, dst_r, zeros128)
    g2 = _mm2_tc(acc1, g1, dinv, b1, W2)
    acc2 = _make_scatter_sc()(g2, src_r, dst_r, zeros128)
    h, z = _mm3_tc(acc2, g2, dinv, b2, We, be)
    y = _make_edge_sc()(z.reshape(NP // 128, 128), src_r, dst_r)
    return (h[:N], y.reshape(-1)[:E, None])
